# baseline (speedup 1.0000x reference)
import math

import jax
import jax.numpy as jnp
import numpy as np
from jax import lax
from jax.experimental import pallas as pl
from jax.experimental.pallas import tpu as pltpu
from jax.experimental.pallas import tpu_sc as plsc

VOCAB = 100000
EMBED = 128
SEQ = 200
BATCH = 1024
HALF = 100
NC, NS = 2, 16
NW = NC * NS
SEQ_PER_W = BATCH // NW
CH_PER_W = SEQ_PER_W * 2
NBUF = 4


def _pe_table():
    pos = np.arange(SEQ, dtype=np.float32)[:, None]
    div = np.exp(
        np.arange(0, EMBED, 2, dtype=np.float32) * -(math.log(10000.0) / EMBED)
    )
    pe = np.zeros((SEQ, EMBED), dtype=np.float32)
    pe[:, 0::2] = np.sin(pos * div)
    pe[:, 1::2] = np.cos(pos * div)
    return pe


_PE = _pe_table()


def _body(idx_hbm, table_hbm, pe_hbm, out_hbm,
          idx_v, pe_v, bufs, isem, gsem, osem):
    wid = lax.axis_index("s") * NC + lax.axis_index("c")
    pltpu.sync_copy(pe_hbm, pe_v)
    ch0 = wid * CH_PER_W
    row0 = wid * SEQ_PER_W * SEQ

    idxd, gathd, outd = {}, {}, {}

    def start_idx(s):
        b = s % NBUF
        idxd[s] = pltpu.async_copy(
            idx_hbm.at[pl.ds(ch0 + 2 * s, 2)], idx_v.at[b], isem.at[b]
        )

    def start_gathers(s):
        b = s % NBUF
        idxd.pop(s).wait()
        gathd[s] = [
            pltpu.async_copy(
                table_hbm.at[idx_v.at[b, h]],
                bufs.at[b, pl.ds(h * HALF, HALF)],
                gsem.at[b],
            )
            for h in range(2)
        ]

    def add_and_out(s):
        b = s % NBUF
        for d in gathd.pop(s):
            d.wait()
        if s + NBUF < SEQ_PER_W:
            start_idx(s + NBUF)

        def add_row(r, _):
            for j in range(8):
                sl = pl.ds(j * 16, 16)
                bufs[b, r, sl] = bufs[b, r, sl] + pe_v[r, sl]
            return 0

        lax.fori_loop(0, SEQ, add_row, 0)
        outd[s] = pltpu.async_copy(
            bufs.at[b], out_hbm.at[pl.ds(row0 + s * SEQ, SEQ)], osem.at[b]
        )

    for s in range(NBUF):
        start_idx(s)
    for s in range(2):
        start_gathers(s)
    for i in range(SEQ_PER_W):
        add_and_out(i)
        if i + 2 < SEQ_PER_W:
            if i >= 2:
                outd.pop(i - 2).wait()
            start_gathers(i + 2)
    for s in sorted(outd):
        outd[s].wait()


def kernel(token_seq, token_table):
    idx = token_seq.astype(jnp.int32).reshape(BATCH * 2, HALF)
    pe = jnp.asarray(_PE)
    f = pl.kernel(
        _body,
        out_type=jax.ShapeDtypeStruct((BATCH * SEQ, EMBED), jnp.float32),
        mesh=plsc.VectorSubcoreMesh(core_axis_name="c", subcore_axis_name="s"),
        scratch_types=[
            pltpu.VMEM((NBUF, 2, HALF), jnp.int32),
            pltpu.VMEM((SEQ, EMBED), jnp.float32),
            pltpu.VMEM((NBUF, SEQ, EMBED), jnp.float32),
            pltpu.SemaphoreType.DMA((NBUF,)),
            pltpu.SemaphoreType.DMA((NBUF,)),
            pltpu.SemaphoreType.DMA((NBUF,)),
        ],
    )
    out = f(idx, token_table, pe)
    return out.reshape(BATCH, SEQ, EMBED)


# speedup vs baseline: 1.0127x; 1.0127x over previous
import math

import jax
import jax.numpy as jnp
import numpy as np
from jax import lax
from jax.experimental import pallas as pl
from jax.experimental.pallas import tpu as pltpu
from jax.experimental.pallas import tpu_sc as plsc

VOCAB = 100000
EMBED = 128
SEQ = 200
BATCH = 1024
HALF = 100
NC, NS = 2, 16
NW = NC * NS
SEQ_PER_W = BATCH // NW
CH_PER_W = SEQ_PER_W * 2
NBUF = 3


def _pe_table():
    pos = np.arange(SEQ, dtype=np.float32)[:, None]
    div = np.exp(
        np.arange(0, EMBED, 2, dtype=np.float32) * -(math.log(10000.0) / EMBED)
    )
    pe = np.zeros((SEQ, EMBED), dtype=np.float32)
    pe[:, 0::2] = np.sin(pos * div)
    pe[:, 1::2] = np.cos(pos * div)
    return pe


_PE = _pe_table()


def _body(idx_hbm, table_hbm, pe_hbm, out_hbm,
          idx_v, pe_v, bufs, gsem, osem, ssem, psem):
    wid = lax.axis_index("s") * NC + lax.axis_index("c")
    ch0 = wid * CH_PER_W
    headd = pltpu.async_copy(
        idx_hbm.at[pl.ds(ch0, 8)], idx_v.at[pl.ds(0, 8)], ssem
    )
    restd = pltpu.async_copy(
        idx_hbm.at[pl.ds(ch0 + 8, CH_PER_W - 8)],
        idx_v.at[pl.ds(8, CH_PER_W - 8)],
        ssem,
    )
    ped = pltpu.async_copy(pe_hbm, pe_v, psem)
    row0 = wid * SEQ_PER_W * SEQ

    gathd, outd = {}, {}

    def start_gathers(s):
        b = s % NBUF
        gathd[s] = [
            pltpu.async_copy(
                table_hbm.at[idx_v.at[s * 2 + h]],
                bufs.at[b, pl.ds(h * HALF, HALF)],
                gsem.at[b],
            )
            for h in range(2)
        ]

    def add_and_out(s):
        b = s % NBUF
        for d in gathd.pop(s):
            d.wait()
        if s == 0:
            ped.wait()

        def add_row(r, _):
            for j in range(8):
                sl = pl.ds(j * 16, 16)
                bufs[b, r, sl] = bufs[b, r, sl] + pe_v[r, sl]
            return 0

        lax.fori_loop(0, SEQ, add_row, 0)

        outd[s] = pltpu.async_copy(
            bufs.at[b], out_hbm.at[pl.ds(row0 + s * SEQ, SEQ)], osem.at[b]
        )

    headd.wait()
    start_gathers(0)
    start_gathers(1)
    restd.wait()
    for i in range(SEQ_PER_W):
        add_and_out(i)
        if i + 2 < SEQ_PER_W:
            if i >= 1:
                outd.pop(i - 1).wait()
            start_gathers(i + 2)
    for s in sorted(outd):
        outd[s].wait()


def kernel(token_seq, token_table):
    idx = token_seq.astype(jnp.int32).reshape(BATCH * 2, HALF)
    pe = jnp.asarray(_PE)
    f = pl.kernel(
        _body,
        out_type=jax.ShapeDtypeStruct((BATCH * SEQ, EMBED), jnp.float32),
        mesh=plsc.VectorSubcoreMesh(core_axis_name="c", subcore_axis_name="s"),
        scratch_types=[
            pltpu.VMEM((CH_PER_W, HALF), jnp.int32),
            pltpu.VMEM((SEQ, EMBED), jnp.float32),
            pltpu.VMEM((NBUF, SEQ, EMBED), jnp.float32),
            pltpu.SemaphoreType.DMA((NBUF,)),
            pltpu.SemaphoreType.DMA((NBUF,)),
            pltpu.SemaphoreType.DMA,
            pltpu.SemaphoreType.DMA,
        ],
    )
    out = f(idx, token_table, pe)
    return out.reshape(BATCH, SEQ, EMBED)


# speedup vs baseline: 1.0133x; 1.0005x over previous
import math

import jax
import jax.numpy as jnp
import numpy as np
from jax import lax
from jax.experimental import pallas as pl
from jax.experimental.pallas import tpu as pltpu
from jax.experimental.pallas import tpu_sc as plsc

VOCAB = 100000
EMBED = 128
SEQ = 200
BATCH = 1024
HALF = 100
NC, NS = 2, 16
NW = NC * NS
SEQ_PER_W = BATCH // NW
CH_PER_W = SEQ_PER_W * 2
NBUF = 3


def _pe_table():
    pos = np.arange(SEQ, dtype=np.float32)[:, None]
    div = np.exp(
        np.arange(0, EMBED, 2, dtype=np.float32) * -(math.log(10000.0) / EMBED)
    )
    pe = np.zeros((SEQ, EMBED), dtype=np.float32)
    pe[:, 0::2] = np.sin(pos * div)
    pe[:, 1::2] = np.cos(pos * div)
    return pe


_PE = _pe_table()


def _body(idx_hbm, table_hbm, pe_hbm, out_hbm,
          idx_v, pe_v, bufs, gsem, osem, ssem, psem):
    wid = lax.axis_index("s") * NC + lax.axis_index("c")
    ch0 = wid * CH_PER_W
    headd = pltpu.async_copy(
        idx_hbm.at[pl.ds(ch0, 8)], idx_v.at[pl.ds(0, 8)], ssem
    )
    restd = pltpu.async_copy(
        idx_hbm.at[pl.ds(ch0 + 8, CH_PER_W - 8)],
        idx_v.at[pl.ds(8, CH_PER_W - 8)],
        ssem,
    )
    ped = pltpu.async_copy(pe_hbm, pe_v, psem)
    row0 = wid * SEQ_PER_W * SEQ

    gathd, outd = {}, {}

    def start_gathers(s):
        b = s % NBUF
        gathd[s] = [
            pltpu.async_copy(
                table_hbm.at[idx_v.at[s * 2 + h]],
                bufs.at[b, pl.ds(h * HALF, HALF)],
                gsem.at[b],
            )
            for h in range(2)
        ]

    def add_and_out(s):
        b = s % NBUF
        for d in gathd.pop(s):
            d.wait()
        if s == 0:
            ped.wait()

        def add_row(r, _):
            for j in range(8):
                sl = pl.ds(j * 16, 16)
                bufs[b, r, sl] = bufs[b, r, sl] + pe_v[r, sl]
            return 0

        lax.fori_loop(0, SEQ, add_row, 0)

        outd[s] = pltpu.async_copy(
            bufs.at[b], out_hbm.at[pl.ds(row0 + s * SEQ, SEQ)], osem.at[b]
        )

    headd.wait()
    start_gathers(0)
    start_gathers(1)
    restd.wait()
    for i in range(SEQ_PER_W):
        add_and_out(i)
        if i + 2 < SEQ_PER_W:
            if i >= 1:
                outd.pop(i - 1).wait()
            start_gathers(i + 2)
    for s in sorted(outd):
        outd[s].wait()


def kernel(token_seq, token_table):
    idx = token_seq.astype(jnp.int32).reshape(BATCH * 2, HALF)
    pe = jnp.asarray(_PE)
    f = pl.kernel(
        _body,
        out_type=jax.ShapeDtypeStruct((BATCH * SEQ, EMBED), jnp.float32),
        mesh=plsc.VectorSubcoreMesh(
            core_axis_name="c", subcore_axis_name="s",
            num_cores=NC, num_subcores=NS,
        ),
        scratch_types=[
            pltpu.VMEM((CH_PER_W, HALF), jnp.int32),
            pltpu.VMEM((SEQ, EMBED), jnp.float32),
            pltpu.VMEM((NBUF, SEQ, EMBED), jnp.float32),
            pltpu.SemaphoreType.DMA((NBUF,)),
            pltpu.SemaphoreType.DMA((NBUF,)),
            pltpu.SemaphoreType.DMA,
            pltpu.SemaphoreType.DMA,
        ],
    )
    out = f(idx, token_table, pe)
    return out.reshape(BATCH, SEQ, EMBED)
